# baseline (speedup 1.0000x reference)
import functools

import jax
import jax.numpy as jnp
from jax import lax
from jax.experimental import pallas as pl
from jax.experimental.pallas import tpu as pltpu
from jax.experimental.pallas import tpu_sc as plsc

USER_NUM = 55485
ITEM_NUM = 5986
N = USER_NUM + ITEM_NUM
D = 128
E = 600000

NC, NS = 2, 16
IDXW = 256
EP = 622592
EROWS = EP // IDXW
RPT = EROWS // NS
W = 16
G = 2
HALF = RPT // 2
PAIRS = HALF // (2 * G)
RPT32 = EROWS // (NC * NS)
CB = 4
N_ACC = 63488
RPS = N_ACC // NS
NCNT = 65536
CRPS = NCNT // NS

_HI = jax.lax.Precision.HIGHEST


def _leaky(v):
    return jnp.where(v > 0, v, 0.01 * v)


def _sc_count_body(dstp, z1d, ones128, out, dstv, ones_v, csem, acc_cnt):
    cid = lax.axis_index("c")
    sid = lax.axis_index("s")
    my_cnt = pl.ds(sid * CRPS, CRPS)
    rbase = (cid * NS + sid) * RPT32
    pltpu.sync_copy(ones128, ones_v)
    pltpu.sync_copy(z1d, acc_cnt.at[my_cnt])
    pltpu.sync_copy(dstp.at[pl.ds(rbase, RPT32)], dstv)
    plsc.subcore_barrier()
    prev = None
    for batch in range(RPT32 // CB):
        cur = [pltpu.async_copy(ones_v, acc_cnt.at[dstv.at[batch * CB + b]],
                                csem, add=True) for b in range(CB)]
        if prev is not None:
            for d in prev:
                d.wait()
        prev = cur
    for d in prev:
        d.wait()
    plsc.subcore_barrier()
    pltpu.sync_copy(acc_cnt.at[my_cnt],
                    out.at[pl.ds(cid * NCNT + sid * CRPS, CRPS)])


def _sc_count(dstp):
    mesh = plsc.VectorSubcoreMesh(core_axis_name="c", subcore_axis_name="s",
                                  num_cores=NC, num_subcores=NS)
    f = pl.kernel(
        _sc_count_body,
        out_type=jax.ShapeDtypeStruct((2 * NCNT,), jnp.float32),
        mesh=mesh,
        scratch_types=[
            pltpu.VMEM((RPT32, IDXW), jnp.int32),
            pltpu.VMEM((IDXW,), jnp.float32),
            pltpu.SemaphoreType.DMA,
            pltpu.VMEM_SHARED((NCNT,), jnp.float32),
        ],
        compiler_params=pltpu.CompilerParams(use_tc_tiling_on_sc=False),
    )
    return f(dstp, jnp.zeros((CRPS,), jnp.float32),
             jnp.ones((IDXW,), jnp.float32))


def _sc_agg_body(nchunk, table, srcp, dstp, zrows, out,
                 srcv1, srcv2, dstv, bufa, bufb,
                 gsema, gsemb, ssema, ssemb, isem1, isem2, acc):
    cid = lax.axis_index("c")
    sid = lax.axis_index("s")
    base = sid * RPT
    my_rows = pl.ds(sid * RPS, RPS)
    per_core = nchunk // NC

    def fire_gathers(srcv, off, bufs, sem):
        return [pltpu.async_copy(table.at[srcv.at[off + b]], bufs.at[b], sem)
                for b in range(G)]

    def fire_scatters(r, bufs, sem):
        return [pltpu.async_copy(bufs.at[b], acc.at[dstv.at[r * G + b]],
                                 sem, add=True) for b in range(G)]

    def chunk_body(cc, carry):
        c = cid * per_core + cc
        crow = c * EROWS

        pltpu.sync_copy(zrows, acc.at[my_rows])
        plsc.subcore_barrier()

        def half_body(h, carry2):
            hbase = base + h * HALF
            pltpu.sync_copy(dstp.at[pl.ds(hbase, HALF)], dstv)
            srow = crow + hbase
            pltpu.sync_copy(srcp.at[pl.ds(srow, 2 * G)], srcv1)
            ip = [None,
                  pltpu.async_copy(srcp.at[pl.ds(srow + 2 * G, 2 * G)],
                                   srcv2, isem2)]
            sa_prev = sb_prev = None
            for t in range(PAIRS):
                sv = srcv1 if t % 2 == 0 else srcv2
                if ip[t % 2] is not None:
                    ip[t % 2].wait()
                    ip[t % 2] = None
                if sa_prev is not None:
                    for d in sa_prev:
                        d.wait()
                ga = fire_gathers(sv, 0, bufa, gsema)
                if sb_prev is not None:
                    for d in sb_prev:
                        d.wait()
                gb = fire_gathers(sv, G, bufb, gsemb)
                for d in ga:
                    d.wait()
                sa_prev = fire_scatters(2 * t, bufa, ssema)
                for d in gb:
                    d.wait()
                if t + 2 < PAIRS:
                    ip[t % 2] = pltpu.async_copy(
                        srcp.at[pl.ds(srow + (t + 2) * 2 * G, 2 * G)],
                        sv, isem1 if t % 2 == 0 else isem2)
                sb_prev = fire_scatters(2 * t + 1, bufb, ssemb)
            for d in sa_prev:
                d.wait()
            for d in sb_prev:
                d.wait()
            return carry2

        lax.fori_loop(0, 2, half_body, 0)
        plsc.subcore_barrier()
        pltpu.sync_copy(acc.at[my_rows], out.at[my_rows, pl.ds(c * W, W)])
        return carry

    lax.fori_loop(0, per_core, chunk_body, 0)


def _sc_agg(nchunk, table, srcp, dstp):
    zrows = jnp.zeros((RPS, W), jnp.float32)
    mesh = plsc.VectorSubcoreMesh(core_axis_name="c", subcore_axis_name="s",
                                  num_cores=NC, num_subcores=NS)
    f = pl.kernel(
        functools.partial(_sc_agg_body, nchunk),
        out_type=jax.ShapeDtypeStruct((N_ACC, nchunk * W), jnp.float32),
        mesh=mesh,
        scratch_types=[
            pltpu.VMEM((2 * G, IDXW), jnp.int32),
            pltpu.VMEM((2 * G, IDXW), jnp.int32),
            pltpu.VMEM((HALF, IDXW), jnp.int32),
            pltpu.VMEM((G, IDXW, W), jnp.float32),
            pltpu.VMEM((G, IDXW, W), jnp.float32),
            pltpu.SemaphoreType.DMA,
            pltpu.SemaphoreType.DMA,
            pltpu.SemaphoreType.DMA,
            pltpu.SemaphoreType.DMA,
            pltpu.SemaphoreType.DMA,
            pltpu.SemaphoreType.DMA,
            pltpu.VMEM_SHARED((N_ACC, W), jnp.float32),
        ],
        compiler_params=pltpu.CompilerParams(use_tc_tiling_on_sc=False),
    )
    return f(table, srcp, dstp, zrows)


_BLK = 2048


def _fuse_body(v_ref, a_ref, t_ref, id_ref, wv_ref, wa_ref, wt_ref, b_ref,
               o_ref):
    acc = jnp.dot(v_ref[...], wv_ref[...], preferred_element_type=jnp.float32,
                  precision=_HI)
    acc += jnp.dot(a_ref[...], wa_ref[...], preferred_element_type=jnp.float32,
                   precision=_HI)
    acc += jnp.dot(t_ref[...], wt_ref[...], preferred_element_type=jnp.float32,
                   precision=_HI)
    acc += id_ref[...] + b_ref[...]
    o_ref[...] = _leaky(acc)


def _tc_fuse(video, audio, title, id_emb, Wv, Wa, Wt, b_f):
    m = ITEM_NUM
    grid = (pl.cdiv(m, _BLK),)
    row = pl.BlockSpec((_BLK, D), lambda i: (i, 0))
    full = pl.BlockSpec((D, D), lambda i: (0, 0))
    return pl.pallas_call(
        _fuse_body,
        grid=grid,
        in_specs=[row, row, row, row, full, full, full,
                  pl.BlockSpec((1, D), lambda i: (0, 0))],
        out_specs=row,
        out_shape=jax.ShapeDtypeStruct((m, D), jnp.float32),
    )(video, audio, title, id_emb, Wv, Wa, Wt, b_f.reshape(1, D))


def _layer1_body(agg_ref, x_ref, ca_ref, cb_ref, w1l, w1r, b1_ref, w2l, w2r,
                 h2_ref, r2_ref):
    inv = 1.0 / jnp.maximum(ca_ref[...] + cb_ref[...], 1.0)
    acc = jnp.dot(agg_ref[...], w1l[...], preferred_element_type=jnp.float32,
                  precision=_HI) * inv
    acc += jnp.dot(x_ref[...], w1r[...], preferred_element_type=jnp.float32,
                   precision=_HI)
    x1v = _leaky(acc + b1_ref[...])
    h2_ref[...] = jnp.dot(x1v, w2l[...], preferred_element_type=jnp.float32,
                          precision=_HI)
    r2_ref[...] = jnp.dot(x1v, w2r[...], preferred_element_type=jnp.float32,
                          precision=_HI)


def _tc_layer1(agg, x, ca, cb, W1l, W1r, b1, W2l, W2r):
    grid = (pl.cdiv(N, _BLK),)
    rowd = pl.BlockSpec((_BLK, D), lambda i: (i, 0))
    row64 = pl.BlockSpec((_BLK, 64), lambda i: (i, 0))
    return pl.pallas_call(
        _layer1_body,
        grid=grid,
        in_specs=[rowd, rowd,
                  pl.BlockSpec((_BLK, 1), lambda i: (i, 0)),
                  pl.BlockSpec((_BLK, 1), lambda i: (i, 0)),
                  pl.BlockSpec((D, D), lambda i: (0, 0)),
                  pl.BlockSpec((D, D), lambda i: (0, 0)),
                  pl.BlockSpec((1, D), lambda i: (0, 0)),
                  pl.BlockSpec((D, 64), lambda i: (0, 0)),
                  pl.BlockSpec((D, 64), lambda i: (0, 0))],
        out_specs=[row64, row64],
        out_shape=[jax.ShapeDtypeStruct((N, 64), jnp.float32),
                   jax.ShapeDtypeStruct((N, 64), jnp.float32)],
    )(agg, x, ca, cb, W1l, W1r, b1.reshape(1, D), W2l, W2r)


def _final_body(a2_ref, ca_ref, cb_ref, r2_ref, b2_ref, o_ref):
    inv = 1.0 / jnp.maximum(ca_ref[...] + cb_ref[...], 1.0)
    o_ref[...] = a2_ref[...] * inv + r2_ref[...] + b2_ref[...]


def _tc_final(agg2, ca, cb, r2, b2):
    grid = (pl.cdiv(N, _BLK),)
    row64 = pl.BlockSpec((_BLK, 64), lambda i: (i, 0))
    return pl.pallas_call(
        _final_body,
        grid=grid,
        in_specs=[row64,
                  pl.BlockSpec((_BLK, 1), lambda i: (i, 0)),
                  pl.BlockSpec((_BLK, 1), lambda i: (i, 0)),
                  row64,
                  pl.BlockSpec((1, 64), lambda i: (0, 0))],
        out_specs=row64,
        out_shape=jax.ShapeDtypeStruct((N, 64), jnp.float32),
    )(agg2, ca, cb, r2, b2.reshape(1, 64))


def kernel(video, audio, title, edge_index, user, Wv, Wa, Wt, b_f, id_emb,
           W1l, W1r, b1, W2l, W2r, b2):
    src = edge_index[0]
    dst = edge_index[1]
    zpad = jnp.zeros((EP - E,), jnp.int32)
    srcpad = jnp.concatenate([src, zpad])
    srcp8 = ((srcpad * 8)[None, :]
             + jnp.arange(8, dtype=jnp.int32)[:, None]).reshape(
                 8 * EROWS, IDXW)
    srcp4 = ((srcpad * 4)[None, :]
             + jnp.arange(4, dtype=jnp.int32)[:, None]).reshape(
                 4 * EROWS, IDXW)
    dstp = jnp.concatenate(
        [dst, jnp.full((EP - E,), N, jnp.int32)]).reshape(EROWS, IDXW)

    md = _tc_fuse(video, audio, title, id_emb, Wv, Wa, Wt, b_f)
    x = jnp.concatenate([user, md], axis=0)

    cnt2 = _sc_count(dstp)
    ca = cnt2[:N, None]
    cb = cnt2[NCNT:NCNT + N, None]
    agg = _sc_agg(8, x.reshape(N * 8, W), srcp8, dstp)

    h2, r2 = _tc_layer1(agg, x, ca, cb, W1l, W1r, b1, W2l, W2r)

    agg2 = _sc_agg(4, h2.reshape(N * 4, W), srcp4, dstp)

    return _tc_final(agg2, ca, cb, r2, b2)


# speedup vs baseline: 1.0173x; 1.0173x over previous
import functools

import jax
import jax.numpy as jnp
from jax import lax
from jax.experimental import pallas as pl
from jax.experimental.pallas import tpu as pltpu
from jax.experimental.pallas import tpu_sc as plsc

USER_NUM = 55485
ITEM_NUM = 5986
N = USER_NUM + ITEM_NUM
D = 128
E = 600000

NC, NS = 2, 16
IDXW = 128
EP = 622592
EROWS = EP // IDXW
RPT = EROWS // NS
W = 16
G = 4
HALF = RPT // 2
PAIRS = HALF // (2 * G)
RPT32 = EROWS // (NC * NS)
CB = 4
N_ACC = 63488
RPS = N_ACC // NS
NCNT = 65536
CRPS = NCNT // NS

_HI = jax.lax.Precision.HIGHEST


def _leaky(v):
    return jnp.where(v > 0, v, 0.01 * v)


def _sc_count_body(dstp, z1d, ones128, out, dstv, ones_v, csem, acc_cnt):
    cid = lax.axis_index("c")
    sid = lax.axis_index("s")
    my_cnt = pl.ds(sid * CRPS, CRPS)
    rbase = (cid * NS + sid) * RPT32
    pltpu.sync_copy(ones128, ones_v)
    pltpu.sync_copy(z1d, acc_cnt.at[my_cnt])
    pltpu.sync_copy(dstp.at[pl.ds(rbase, RPT32)], dstv)
    plsc.subcore_barrier()
    prev = None
    for batch in range(RPT32 // CB):
        cur = [pltpu.async_copy(ones_v, acc_cnt.at[dstv.at[batch * CB + b]],
                                csem, add=True) for b in range(CB)]
        if prev is not None:
            for d in prev:
                d.wait()
        prev = cur
    for d in prev:
        d.wait()
    plsc.subcore_barrier()
    pltpu.sync_copy(acc_cnt.at[my_cnt],
                    out.at[pl.ds(cid * NCNT + sid * CRPS, CRPS)])


def _sc_count(dstp):
    mesh = plsc.VectorSubcoreMesh(core_axis_name="c", subcore_axis_name="s",
                                  num_cores=NC, num_subcores=NS)
    f = pl.kernel(
        _sc_count_body,
        out_type=jax.ShapeDtypeStruct((2 * NCNT,), jnp.float32),
        mesh=mesh,
        scratch_types=[
            pltpu.VMEM((RPT32, IDXW), jnp.int32),
            pltpu.VMEM((IDXW,), jnp.float32),
            pltpu.SemaphoreType.DMA,
            pltpu.VMEM_SHARED((NCNT,), jnp.float32),
        ],
        compiler_params=pltpu.CompilerParams(use_tc_tiling_on_sc=False),
    )
    return f(dstp, jnp.zeros((CRPS,), jnp.float32),
             jnp.ones((IDXW,), jnp.float32))


def _sc_agg_body(nchunk, table, srcp, dstp, zrows, out,
                 srcv1, srcv2, dstv, bufa, bufb,
                 gsema, gsemb, ssema, ssemb, isem1, isem2, acc):
    cid = lax.axis_index("c")
    sid = lax.axis_index("s")
    base = sid * RPT
    my_rows = pl.ds(sid * RPS, RPS)
    per_core = nchunk // NC

    def fire_gathers(srcv, off, bufs, sem):
        return [pltpu.async_copy(table.at[srcv.at[off + b]], bufs.at[b], sem)
                for b in range(G)]

    def fire_scatters(r, bufs, sem):
        return [pltpu.async_copy(bufs.at[b], acc.at[dstv.at[r * G + b]],
                                 sem, add=True) for b in range(G)]

    def chunk_body(cc, carry):
        c = cid * per_core + cc
        crow = c * EROWS

        pltpu.sync_copy(zrows, acc.at[my_rows])
        plsc.subcore_barrier()

        def half_body(h, carry2):
            hbase = base + h * HALF
            pltpu.sync_copy(dstp.at[pl.ds(hbase, HALF)], dstv)
            srow = crow + hbase
            pltpu.sync_copy(srcp.at[pl.ds(srow, 2 * G)], srcv1)
            ip = [None,
                  pltpu.async_copy(srcp.at[pl.ds(srow + 2 * G, 2 * G)],
                                   srcv2, isem2)]
            sa_prev = sb_prev = None
            for t in range(PAIRS):
                sv = srcv1 if t % 2 == 0 else srcv2
                if ip[t % 2] is not None:
                    ip[t % 2].wait()
                    ip[t % 2] = None
                if sa_prev is not None:
                    for d in sa_prev:
                        d.wait()
                ga = fire_gathers(sv, 0, bufa, gsema)
                if sb_prev is not None:
                    for d in sb_prev:
                        d.wait()
                gb = fire_gathers(sv, G, bufb, gsemb)
                for d in ga:
                    d.wait()
                sa_prev = fire_scatters(2 * t, bufa, ssema)
                for d in gb:
                    d.wait()
                if t + 2 < PAIRS:
                    ip[t % 2] = pltpu.async_copy(
                        srcp.at[pl.ds(srow + (t + 2) * 2 * G, 2 * G)],
                        sv, isem1 if t % 2 == 0 else isem2)
                sb_prev = fire_scatters(2 * t + 1, bufb, ssemb)
            for d in sa_prev:
                d.wait()
            for d in sb_prev:
                d.wait()
            return carry2

        lax.fori_loop(0, 2, half_body, 0)
        plsc.subcore_barrier()
        pltpu.sync_copy(acc.at[my_rows], out.at[my_rows, pl.ds(c * W, W)])
        return carry

    lax.fori_loop(0, per_core, chunk_body, 0)


def _sc_agg(nchunk, table, srcp, dstp):
    zrows = jnp.zeros((RPS, W), jnp.float32)
    mesh = plsc.VectorSubcoreMesh(core_axis_name="c", subcore_axis_name="s",
                                  num_cores=NC, num_subcores=NS)
    f = pl.kernel(
        functools.partial(_sc_agg_body, nchunk),
        out_type=jax.ShapeDtypeStruct((N_ACC, nchunk * W), jnp.float32),
        mesh=mesh,
        scratch_types=[
            pltpu.VMEM((2 * G, IDXW), jnp.int32),
            pltpu.VMEM((2 * G, IDXW), jnp.int32),
            pltpu.VMEM((HALF, IDXW), jnp.int32),
            pltpu.VMEM((G, IDXW, W), jnp.float32),
            pltpu.VMEM((G, IDXW, W), jnp.float32),
            pltpu.SemaphoreType.DMA,
            pltpu.SemaphoreType.DMA,
            pltpu.SemaphoreType.DMA,
            pltpu.SemaphoreType.DMA,
            pltpu.SemaphoreType.DMA,
            pltpu.SemaphoreType.DMA,
            pltpu.VMEM_SHARED((N_ACC, W), jnp.float32),
        ],
        compiler_params=pltpu.CompilerParams(use_tc_tiling_on_sc=False),
    )
    return f(table, srcp, dstp, zrows)


_BLK = 2048


def _fuse_body(v_ref, a_ref, t_ref, id_ref, wv_ref, wa_ref, wt_ref, b_ref,
               o_ref):
    acc = jnp.dot(v_ref[...], wv_ref[...], preferred_element_type=jnp.float32,
                  precision=_HI)
    acc += jnp.dot(a_ref[...], wa_ref[...], preferred_element_type=jnp.float32,
                   precision=_HI)
    acc += jnp.dot(t_ref[...], wt_ref[...], preferred_element_type=jnp.float32,
                   precision=_HI)
    acc += id_ref[...] + b_ref[...]
    o_ref[...] = _leaky(acc)


def _tc_fuse(video, audio, title, id_emb, Wv, Wa, Wt, b_f):
    m = ITEM_NUM
    grid = (pl.cdiv(m, _BLK),)
    row = pl.BlockSpec((_BLK, D), lambda i: (i, 0))
    full = pl.BlockSpec((D, D), lambda i: (0, 0))
    return pl.pallas_call(
        _fuse_body,
        grid=grid,
        in_specs=[row, row, row, row, full, full, full,
                  pl.BlockSpec((1, D), lambda i: (0, 0))],
        out_specs=row,
        out_shape=jax.ShapeDtypeStruct((m, D), jnp.float32),
    )(video, audio, title, id_emb, Wv, Wa, Wt, b_f.reshape(1, D))


def _layer1_body(agg_ref, x_ref, ca_ref, cb_ref, w1l, w1r, b1_ref, w2l, w2r,
                 h2_ref, r2_ref):
    inv = 1.0 / jnp.maximum(ca_ref[...] + cb_ref[...], 1.0)
    acc = jnp.dot(agg_ref[...], w1l[...], preferred_element_type=jnp.float32,
                  precision=_HI) * inv
    acc += jnp.dot(x_ref[...], w1r[...], preferred_element_type=jnp.float32,
                   precision=_HI)
    x1v = _leaky(acc + b1_ref[...])
    h2_ref[...] = jnp.dot(x1v, w2l[...], preferred_element_type=jnp.float32,
                          precision=_HI)
    r2_ref[...] = jnp.dot(x1v, w2r[...], preferred_element_type=jnp.float32,
                          precision=_HI)


def _tc_layer1(agg, x, ca, cb, W1l, W1r, b1, W2l, W2r):
    grid = (pl.cdiv(N, _BLK),)
    rowd = pl.BlockSpec((_BLK, D), lambda i: (i, 0))
    row64 = pl.BlockSpec((_BLK, 64), lambda i: (i, 0))
    return pl.pallas_call(
        _layer1_body,
        grid=grid,
        in_specs=[rowd, rowd,
                  pl.BlockSpec((_BLK, 1), lambda i: (i, 0)),
                  pl.BlockSpec((_BLK, 1), lambda i: (i, 0)),
                  pl.BlockSpec((D, D), lambda i: (0, 0)),
                  pl.BlockSpec((D, D), lambda i: (0, 0)),
                  pl.BlockSpec((1, D), lambda i: (0, 0)),
                  pl.BlockSpec((D, 64), lambda i: (0, 0)),
                  pl.BlockSpec((D, 64), lambda i: (0, 0))],
        out_specs=[row64, row64],
        out_shape=[jax.ShapeDtypeStruct((N, 64), jnp.float32),
                   jax.ShapeDtypeStruct((N, 64), jnp.float32)],
    )(agg, x, ca, cb, W1l, W1r, b1.reshape(1, D), W2l, W2r)


def _final_body(a2_ref, ca_ref, cb_ref, r2_ref, b2_ref, o_ref):
    inv = 1.0 / jnp.maximum(ca_ref[...] + cb_ref[...], 1.0)
    o_ref[...] = a2_ref[...] * inv + r2_ref[...] + b2_ref[...]


def _tc_final(agg2, ca, cb, r2, b2):
    grid = (pl.cdiv(N, _BLK),)
    row64 = pl.BlockSpec((_BLK, 64), lambda i: (i, 0))
    return pl.pallas_call(
        _final_body,
        grid=grid,
        in_specs=[row64,
                  pl.BlockSpec((_BLK, 1), lambda i: (i, 0)),
                  pl.BlockSpec((_BLK, 1), lambda i: (i, 0)),
                  row64,
                  pl.BlockSpec((1, 64), lambda i: (0, 0))],
        out_specs=row64,
        out_shape=jax.ShapeDtypeStruct((N, 64), jnp.float32),
    )(agg2, ca, cb, r2, b2.reshape(1, 64))


def kernel(video, audio, title, edge_index, user, Wv, Wa, Wt, b_f, id_emb,
           W1l, W1r, b1, W2l, W2r, b2):
    src = edge_index[0]
    dst = edge_index[1]
    zpad = jnp.zeros((EP - E,), jnp.int32)
    srcpad = jnp.concatenate([src, zpad])
    srcp8 = ((srcpad * 8)[None, :]
             + jnp.arange(8, dtype=jnp.int32)[:, None]).reshape(
                 8 * EROWS, IDXW)
    srcp4 = ((srcpad * 4)[None, :]
             + jnp.arange(4, dtype=jnp.int32)[:, None]).reshape(
                 4 * EROWS, IDXW)
    dstp = jnp.concatenate(
        [dst, jnp.full((EP - E,), N, jnp.int32)]).reshape(EROWS, IDXW)

    md = _tc_fuse(video, audio, title, id_emb, Wv, Wa, Wt, b_f)
    x = jnp.concatenate([user, md], axis=0)

    cnt2 = _sc_count(dstp)
    ca = cnt2[:N, None]
    cb = cnt2[NCNT:NCNT + N, None]
    agg = _sc_agg(8, x.reshape(N * 8, W), srcp8, dstp)

    h2, r2 = _tc_layer1(agg, x, ca, cb, W1l, W1r, b1, W2l, W2r)

    agg2 = _sc_agg(4, h2.reshape(N * 4, W), srcp4, dstp)

    return _tc_final(agg2, ca, cb, r2, b2)


# speedup vs baseline: 1.0443x; 1.0265x over previous
import functools

import jax
import jax.numpy as jnp
from jax import lax
from jax.experimental import pallas as pl
from jax.experimental.pallas import tpu as pltpu
from jax.experimental.pallas import tpu_sc as plsc

USER_NUM = 55485
ITEM_NUM = 5986
N = USER_NUM + ITEM_NUM
D = 128
E = 600000

NC, NS = 2, 16
IDXW = 128
EP = 622592
EROWS = EP // IDXW
RPT = EROWS // NS
W = 16
G = 4
HALF = RPT // 2
PAIRS = HALF // (2 * G)
RPT32 = EROWS // (NC * NS)
CB = 4
N_ACC = 63488
RPS = N_ACC // NS
NCNT = 65536
CRPS = NCNT // NS

_HI = jax.lax.Precision.DEFAULT


def _leaky(v):
    return jnp.where(v > 0, v, 0.01 * v)


def _sc_count_body(dstp, z1d, ones128, out, dstv, ones_v, csem, acc_cnt):
    cid = lax.axis_index("c")
    sid = lax.axis_index("s")
    my_cnt = pl.ds(sid * CRPS, CRPS)
    rbase = (cid * NS + sid) * RPT32
    pltpu.sync_copy(ones128, ones_v)
    pltpu.sync_copy(z1d, acc_cnt.at[my_cnt])
    pltpu.sync_copy(dstp.at[pl.ds(rbase, RPT32)], dstv)
    plsc.subcore_barrier()
    prev = None
    for batch in range(RPT32 // CB):
        cur = [pltpu.async_copy(ones_v, acc_cnt.at[dstv.at[batch * CB + b]],
                                csem, add=True) for b in range(CB)]
        if prev is not None:
            for d in prev:
                d.wait()
        prev = cur
    for d in prev:
        d.wait()
    plsc.subcore_barrier()
    pltpu.sync_copy(acc_cnt.at[my_cnt],
                    out.at[pl.ds(cid * NCNT + sid * CRPS, CRPS)])


def _sc_count(dstp):
    mesh = plsc.VectorSubcoreMesh(core_axis_name="c", subcore_axis_name="s",
                                  num_cores=NC, num_subcores=NS)
    f = pl.kernel(
        _sc_count_body,
        out_type=jax.ShapeDtypeStruct((2 * NCNT,), jnp.float32),
        mesh=mesh,
        scratch_types=[
            pltpu.VMEM((RPT32, IDXW), jnp.int32),
            pltpu.VMEM((IDXW,), jnp.float32),
            pltpu.SemaphoreType.DMA,
            pltpu.VMEM_SHARED((NCNT,), jnp.float32),
        ],
        compiler_params=pltpu.CompilerParams(use_tc_tiling_on_sc=False),
    )
    return f(dstp, jnp.zeros((CRPS,), jnp.float32),
             jnp.ones((IDXW,), jnp.float32))


def _sc_agg_body(nchunk, table, srcp, dstp, zrows, out,
                 srcv1, srcv2, dstv, bufa, bufb,
                 gsema, gsemb, ssema, ssemb, isem1, isem2, acc):
    cid = lax.axis_index("c")
    sid = lax.axis_index("s")
    base = sid * RPT
    my_rows = pl.ds(sid * RPS, RPS)
    per_core = nchunk // NC

    def fire_gathers(srcv, off, bufs, sem):
        return [pltpu.async_copy(table.at[srcv.at[off + b]], bufs.at[b], sem)
                for b in range(G)]

    def fire_scatters(r, bufs, sem):
        return [pltpu.async_copy(bufs.at[b], acc.at[dstv.at[r * G + b]],
                                 sem, add=True) for b in range(G)]

    def chunk_body(cc, carry):
        c = cid * per_core + cc
        crow = c * EROWS

        pltpu.sync_copy(zrows, acc.at[my_rows])
        plsc.subcore_barrier()

        def half_body(h, carry2):
            hbase = base + h * HALF
            pltpu.sync_copy(dstp.at[pl.ds(hbase, HALF)], dstv)
            srow = crow + hbase
            pltpu.sync_copy(srcp.at[pl.ds(srow, 2 * G)], srcv1)
            ip = [None,
                  pltpu.async_copy(srcp.at[pl.ds(srow + 2 * G, 2 * G)],
                                   srcv2, isem2)]
            sa_prev = sb_prev = None
            for t in range(PAIRS):
                sv = srcv1 if t % 2 == 0 else srcv2
                if ip[t % 2] is not None:
                    ip[t % 2].wait()
                    ip[t % 2] = None
                if sa_prev is not None:
                    for d in sa_prev:
                        d.wait()
                ga = fire_gathers(sv, 0, bufa, gsema)
                if sb_prev is not None:
                    for d in sb_prev:
                        d.wait()
                gb = fire_gathers(sv, G, bufb, gsemb)
                for d in ga:
                    d.wait()
                sa_prev = fire_scatters(2 * t, bufa, ssema)
                for d in gb:
                    d.wait()
                if t + 2 < PAIRS:
                    ip[t % 2] = pltpu.async_copy(
                        srcp.at[pl.ds(srow + (t + 2) * 2 * G, 2 * G)],
                        sv, isem1 if t % 2 == 0 else isem2)
                sb_prev = fire_scatters(2 * t + 1, bufb, ssemb)
            for d in sa_prev:
                d.wait()
            for d in sb_prev:
                d.wait()
            return carry2

        lax.fori_loop(0, 2, half_body, 0)
        plsc.subcore_barrier()
        pltpu.sync_copy(acc.at[my_rows], out.at[my_rows, pl.ds(c * W, W)])
        return carry

    lax.fori_loop(0, per_core, chunk_body, 0)


def _sc_agg(nchunk, table, srcp, dstp):
    zrows = jnp.zeros((RPS, W), jnp.float32)
    mesh = plsc.VectorSubcoreMesh(core_axis_name="c", subcore_axis_name="s",
                                  num_cores=NC, num_subcores=NS)
    f = pl.kernel(
        functools.partial(_sc_agg_body, nchunk),
        out_type=jax.ShapeDtypeStruct((N_ACC, nchunk * W), jnp.float32),
        mesh=mesh,
        scratch_types=[
            pltpu.VMEM((2 * G, IDXW), jnp.int32),
            pltpu.VMEM((2 * G, IDXW), jnp.int32),
            pltpu.VMEM((HALF, IDXW), jnp.int32),
            pltpu.VMEM((G, IDXW, W), jnp.float32),
            pltpu.VMEM((G, IDXW, W), jnp.float32),
            pltpu.SemaphoreType.DMA,
            pltpu.SemaphoreType.DMA,
            pltpu.SemaphoreType.DMA,
            pltpu.SemaphoreType.DMA,
            pltpu.SemaphoreType.DMA,
            pltpu.SemaphoreType.DMA,
            pltpu.VMEM_SHARED((N_ACC, W), jnp.float32),
        ],
        compiler_params=pltpu.CompilerParams(use_tc_tiling_on_sc=False),
    )
    return f(table, srcp, dstp, zrows)


_BLK = 2048


def _fuse_body(v_ref, a_ref, t_ref, id_ref, wv_ref, wa_ref, wt_ref, b_ref,
               o_ref):
    acc = jnp.dot(v_ref[...], wv_ref[...], preferred_element_type=jnp.float32,
                  precision=_HI)
    acc += jnp.dot(a_ref[...], wa_ref[...], preferred_element_type=jnp.float32,
                   precision=_HI)
    acc += jnp.dot(t_ref[...], wt_ref[...], preferred_element_type=jnp.float32,
                   precision=_HI)
    acc += id_ref[...] + b_ref[...]
    o_ref[...] = _leaky(acc)


def _tc_fuse(video, audio, title, id_emb, Wv, Wa, Wt, b_f):
    m = ITEM_NUM
    grid = (pl.cdiv(m, _BLK),)
    row = pl.BlockSpec((_BLK, D), lambda i: (i, 0))
    full = pl.BlockSpec((D, D), lambda i: (0, 0))
    return pl.pallas_call(
        _fuse_body,
        grid=grid,
        in_specs=[row, row, row, row, full, full, full,
                  pl.BlockSpec((1, D), lambda i: (0, 0))],
        out_specs=row,
        out_shape=jax.ShapeDtypeStruct((m, D), jnp.float32),
    )(video, audio, title, id_emb, Wv, Wa, Wt, b_f.reshape(1, D))


def _layer1_body(agg_ref, x_ref, ca_ref, cb_ref, w1l, w1r, b1_ref, w2l, w2r,
                 h2_ref, r2_ref):
    inv = 1.0 / jnp.maximum(ca_ref[...] + cb_ref[...], 1.0)
    acc = jnp.dot(agg_ref[...], w1l[...], preferred_element_type=jnp.float32,
                  precision=_HI) * inv
    acc += jnp.dot(x_ref[...], w1r[...], preferred_element_type=jnp.float32,
                   precision=_HI)
    x1v = _leaky(acc + b1_ref[...])
    h2_ref[...] = jnp.dot(x1v, w2l[...], preferred_element_type=jnp.float32,
                          precision=_HI)
    r2_ref[...] = jnp.dot(x1v, w2r[...], preferred_element_type=jnp.float32,
                          precision=_HI)


def _tc_layer1(agg, x, ca, cb, W1l, W1r, b1, W2l, W2r):
    grid = (pl.cdiv(N, _BLK),)
    rowd = pl.BlockSpec((_BLK, D), lambda i: (i, 0))
    row64 = pl.BlockSpec((_BLK, 64), lambda i: (i, 0))
    return pl.pallas_call(
        _layer1_body,
        grid=grid,
        in_specs=[rowd, rowd,
                  pl.BlockSpec((_BLK, 1), lambda i: (i, 0)),
                  pl.BlockSpec((_BLK, 1), lambda i: (i, 0)),
                  pl.BlockSpec((D, D), lambda i: (0, 0)),
                  pl.BlockSpec((D, D), lambda i: (0, 0)),
                  pl.BlockSpec((1, D), lambda i: (0, 0)),
                  pl.BlockSpec((D, 64), lambda i: (0, 0)),
                  pl.BlockSpec((D, 64), lambda i: (0, 0))],
        out_specs=[row64, row64],
        out_shape=[jax.ShapeDtypeStruct((N, 64), jnp.float32),
                   jax.ShapeDtypeStruct((N, 64), jnp.float32)],
    )(agg, x, ca, cb, W1l, W1r, b1.reshape(1, D), W2l, W2r)


def _final_body(a2_ref, ca_ref, cb_ref, r2_ref, b2_ref, o_ref):
    inv = 1.0 / jnp.maximum(ca_ref[...] + cb_ref[...], 1.0)
    o_ref[...] = a2_ref[...] * inv + r2_ref[...] + b2_ref[...]


def _tc_final(agg2, ca, cb, r2, b2):
    grid = (pl.cdiv(N, _BLK),)
    row64 = pl.BlockSpec((_BLK, 64), lambda i: (i, 0))
    return pl.pallas_call(
        _final_body,
        grid=grid,
        in_specs=[row64,
                  pl.BlockSpec((_BLK, 1), lambda i: (i, 0)),
                  pl.BlockSpec((_BLK, 1), lambda i: (i, 0)),
                  row64,
                  pl.BlockSpec((1, 64), lambda i: (0, 0))],
        out_specs=row64,
        out_shape=jax.ShapeDtypeStruct((N, 64), jnp.float32),
    )(agg2, ca, cb, r2, b2.reshape(1, 64))


def kernel(video, audio, title, edge_index, user, Wv, Wa, Wt, b_f, id_emb,
           W1l, W1r, b1, W2l, W2r, b2):
    src = edge_index[0]
    dst = edge_index[1]
    zpad = jnp.zeros((EP - E,), jnp.int32)
    srcpad = jnp.concatenate([src, zpad])
    srcp8 = ((srcpad * 8)[None, :]
             + jnp.arange(8, dtype=jnp.int32)[:, None]).reshape(
                 8 * EROWS, IDXW)
    srcp4 = ((srcpad * 4)[None, :]
             + jnp.arange(4, dtype=jnp.int32)[:, None]).reshape(
                 4 * EROWS, IDXW)
    dstp = jnp.concatenate(
        [dst, jnp.full((EP - E,), N, jnp.int32)]).reshape(EROWS, IDXW)

    md = _tc_fuse(video, audio, title, id_emb, Wv, Wa, Wt, b_f)
    x = jnp.concatenate([user, md], axis=0)

    cnt2 = _sc_count(dstp)
    ca = cnt2[:N, None]
    cb = cnt2[NCNT:NCNT + N, None]
    agg = _sc_agg(8, x.reshape(N * 8, W), srcp8, dstp)

    h2, r2 = _tc_layer1(agg, x, ca, cb, W1l, W1r, b1, W2l, W2r)

    agg2 = _sc_agg(4, h2.reshape(N * 4, W), srcp4, dstp)

    return _tc_final(agg2, ca, cb, r2, b2)


# speedup vs baseline: 1.0497x; 1.0052x over previous
import functools

import jax
import jax.numpy as jnp
from jax import lax
from jax.experimental import pallas as pl
from jax.experimental.pallas import tpu as pltpu
from jax.experimental.pallas import tpu_sc as plsc

USER_NUM = 55485
ITEM_NUM = 5986
N = USER_NUM + ITEM_NUM
D = 128
E = 600000

NC, NS = 2, 16
IDXW = 128
EP = 622592
EROWS = EP // IDXW
RPT = EROWS // NS
W = 16
G = 4
HALF = RPT // 2
PAIRS = HALF // (2 * G)
RPT32 = EROWS // (NC * NS)
CB = 4
N_ACC = 63488
RPS = N_ACC // NS
NCNT = 65536
CRPS = NCNT // NS

_HI = jax.lax.Precision.DEFAULT


def _leaky(v):
    return jnp.where(v > 0, v, 0.01 * v)


def _sc_count_body(dstp, z1d, ones128, out, dstv, ones_v, csem, acc_cnt):
    cid = lax.axis_index("c")
    sid = lax.axis_index("s")
    my_cnt = pl.ds(sid * CRPS, CRPS)
    rbase = (cid * NS + sid) * RPT32
    pltpu.sync_copy(ones128, ones_v)
    pltpu.sync_copy(z1d, acc_cnt.at[my_cnt])
    pltpu.sync_copy(dstp.at[pl.ds(rbase, RPT32)], dstv)
    plsc.subcore_barrier()
    prev = None
    for batch in range(RPT32 // CB):
        cur = [pltpu.async_copy(ones_v, acc_cnt.at[dstv.at[batch * CB + b]],
                                csem, add=True) for b in range(CB)]
        if prev is not None:
            for d in prev:
                d.wait()
        prev = cur
    for d in prev:
        d.wait()
    plsc.subcore_barrier()
    pltpu.sync_copy(acc_cnt.at[my_cnt],
                    out.at[pl.ds(cid * NCNT + sid * CRPS, CRPS)])


def _sc_count(dstp):
    mesh = plsc.VectorSubcoreMesh(core_axis_name="c", subcore_axis_name="s",
                                  num_cores=NC, num_subcores=NS)
    f = pl.kernel(
        _sc_count_body,
        out_type=jax.ShapeDtypeStruct((2 * NCNT,), jnp.float32),
        mesh=mesh,
        scratch_types=[
            pltpu.VMEM((RPT32, IDXW), jnp.int32),
            pltpu.VMEM((IDXW,), jnp.float32),
            pltpu.SemaphoreType.DMA,
            pltpu.VMEM_SHARED((NCNT,), jnp.float32),
        ],
        compiler_params=pltpu.CompilerParams(use_tc_tiling_on_sc=False),
    )
    return f(dstp, jnp.zeros((CRPS,), jnp.float32),
             jnp.ones((IDXW,), jnp.float32))


def _sc_agg_body(nchunk, table, srcp, dstp, zrows, out,
                 srcv1, srcv2, dstv, bufa, bufb,
                 gsema, gsemb, ssema, ssemb, isem1, isem2, acc):
    cid = lax.axis_index("c")
    sid = lax.axis_index("s")
    base = sid * RPT
    my_rows = pl.ds(sid * RPS, RPS)
    per_core = nchunk // NC

    def fire_gathers(srcv, off, bufs, sem):
        return [pltpu.async_copy(table.at[srcv.at[off + b]], bufs.at[b], sem)
                for b in range(G)]

    def fire_scatters(r, bufs, sem):
        return [pltpu.async_copy(bufs.at[b], acc.at[dstv.at[r * G + b]],
                                 sem, add=True) for b in range(G)]

    def chunk_body(cc, carry):
        c = cid * per_core + cc
        crow = c * EROWS

        pltpu.sync_copy(zrows, acc.at[my_rows])
        plsc.subcore_barrier()

        def half_body(h, carry2):
            hbase = base + h * HALF
            pltpu.sync_copy(dstp.at[pl.ds(hbase, HALF)], dstv)
            srow = crow + hbase
            pltpu.sync_copy(srcp.at[pl.ds(srow, 2 * G)], srcv1)
            ip = [None,
                  pltpu.async_copy(srcp.at[pl.ds(srow + 2 * G, 2 * G)],
                                   srcv2, isem2)]
            sa_prev = sb_prev = None
            for t in range(PAIRS):
                sv = srcv1 if t % 2 == 0 else srcv2
                if ip[t % 2] is not None:
                    ip[t % 2].wait()
                    ip[t % 2] = None
                if sa_prev is not None:
                    for d in sa_prev:
                        d.wait()
                ga = fire_gathers(sv, 0, bufa, gsema)
                if sb_prev is not None:
                    for d in sb_prev:
                        d.wait()
                gb = fire_gathers(sv, G, bufb, gsemb)
                for d in ga:
                    d.wait()
                sa_prev = fire_scatters(2 * t, bufa, ssema)
                for d in gb:
                    d.wait()
                if t + 2 < PAIRS:
                    ip[t % 2] = pltpu.async_copy(
                        srcp.at[pl.ds(srow + (t + 2) * 2 * G, 2 * G)],
                        sv, isem1 if t % 2 == 0 else isem2)
                sb_prev = fire_scatters(2 * t + 1, bufb, ssemb)
            for d in sa_prev:
                d.wait()
            for d in sb_prev:
                d.wait()
            return carry2

        lax.fori_loop(0, 2, half_body, 0)
        plsc.subcore_barrier()
        pltpu.sync_copy(acc.at[my_rows], out.at[my_rows, pl.ds(c * W, W)])
        return carry

    lax.fori_loop(0, per_core, chunk_body, 0)


def _sc_agg(nchunk, table, srcp, dstp):
    zrows = jnp.zeros((RPS, W), jnp.float32)
    mesh = plsc.VectorSubcoreMesh(core_axis_name="c", subcore_axis_name="s",
                                  num_cores=NC, num_subcores=NS)
    f = pl.kernel(
        functools.partial(_sc_agg_body, nchunk),
        out_type=jax.ShapeDtypeStruct((N_ACC, nchunk * W), jnp.float32),
        mesh=mesh,
        scratch_types=[
            pltpu.VMEM((2 * G, IDXW), jnp.int32),
            pltpu.VMEM((2 * G, IDXW), jnp.int32),
            pltpu.VMEM((HALF, IDXW), jnp.int32),
            pltpu.VMEM((G, IDXW, W), jnp.float32),
            pltpu.VMEM((G, IDXW, W), jnp.float32),
            pltpu.SemaphoreType.DMA,
            pltpu.SemaphoreType.DMA,
            pltpu.SemaphoreType.DMA,
            pltpu.SemaphoreType.DMA,
            pltpu.SemaphoreType.DMA,
            pltpu.SemaphoreType.DMA,
            pltpu.VMEM_SHARED((N_ACC, W), jnp.float32),
        ],
        compiler_params=pltpu.CompilerParams(use_tc_tiling_on_sc=False),
    )
    return f(table, srcp, dstp, zrows)


_BLK = 4096


def _fuse_body(v_ref, a_ref, t_ref, id_ref, wv_ref, wa_ref, wt_ref, b_ref,
               o_ref):
    acc = jnp.dot(v_ref[...], wv_ref[...], preferred_element_type=jnp.float32,
                  precision=_HI)
    acc += jnp.dot(a_ref[...], wa_ref[...], preferred_element_type=jnp.float32,
                   precision=_HI)
    acc += jnp.dot(t_ref[...], wt_ref[...], preferred_element_type=jnp.float32,
                   precision=_HI)
    acc += id_ref[...] + b_ref[...]
    o_ref[...] = _leaky(acc)


def _tc_fuse(video, audio, title, id_emb, Wv, Wa, Wt, b_f):
    m = ITEM_NUM
    grid = (pl.cdiv(m, _BLK),)
    row = pl.BlockSpec((_BLK, D), lambda i: (i, 0))
    full = pl.BlockSpec((D, D), lambda i: (0, 0))
    return pl.pallas_call(
        _fuse_body,
        grid=grid,
        in_specs=[row, row, row, row, full, full, full,
                  pl.BlockSpec((1, D), lambda i: (0, 0))],
        out_specs=row,
        out_shape=jax.ShapeDtypeStruct((m, D), jnp.float32),
    )(video, audio, title, id_emb, Wv, Wa, Wt, b_f.reshape(1, D))


def _layer1_body(agg_ref, x_ref, ca_ref, cb_ref, w1l, w1r, b1_ref, w2l, w2r,
                 h2_ref, r2_ref):
    inv = 1.0 / jnp.maximum(ca_ref[...] + cb_ref[...], 1.0)
    acc = jnp.dot(agg_ref[...], w1l[...], preferred_element_type=jnp.float32,
                  precision=_HI) * inv
    acc += jnp.dot(x_ref[...], w1r[...], preferred_element_type=jnp.float32,
                   precision=_HI)
    x1v = _leaky(acc + b1_ref[...])
    h2_ref[...] = jnp.dot(x1v, w2l[...], preferred_element_type=jnp.float32,
                          precision=_HI)
    r2_ref[...] = jnp.dot(x1v, w2r[...], preferred_element_type=jnp.float32,
                          precision=_HI)


def _tc_layer1(agg, x, ca, cb, W1l, W1r, b1, W2l, W2r):
    grid = (pl.cdiv(N, _BLK),)
    rowd = pl.BlockSpec((_BLK, D), lambda i: (i, 0))
    row64 = pl.BlockSpec((_BLK, 64), lambda i: (i, 0))
    return pl.pallas_call(
        _layer1_body,
        grid=grid,
        in_specs=[rowd, rowd,
                  pl.BlockSpec((_BLK, 1), lambda i: (i, 0)),
                  pl.BlockSpec((_BLK, 1), lambda i: (i, 0)),
                  pl.BlockSpec((D, D), lambda i: (0, 0)),
                  pl.BlockSpec((D, D), lambda i: (0, 0)),
                  pl.BlockSpec((1, D), lambda i: (0, 0)),
                  pl.BlockSpec((D, 64), lambda i: (0, 0)),
                  pl.BlockSpec((D, 64), lambda i: (0, 0))],
        out_specs=[row64, row64],
        out_shape=[jax.ShapeDtypeStruct((N, 64), jnp.float32),
                   jax.ShapeDtypeStruct((N, 64), jnp.float32)],
    )(agg, x, ca, cb, W1l, W1r, b1.reshape(1, D), W2l, W2r)


def _final_body(a2_ref, ca_ref, cb_ref, r2_ref, b2_ref, o_ref):
    inv = 1.0 / jnp.maximum(ca_ref[...] + cb_ref[...], 1.0)
    o_ref[...] = a2_ref[...] * inv + r2_ref[...] + b2_ref[...]


def _tc_final(agg2, ca, cb, r2, b2):
    grid = (pl.cdiv(N, _BLK),)
    row64 = pl.BlockSpec((_BLK, 64), lambda i: (i, 0))
    return pl.pallas_call(
        _final_body,
        grid=grid,
        in_specs=[row64,
                  pl.BlockSpec((_BLK, 1), lambda i: (i, 0)),
                  pl.BlockSpec((_BLK, 1), lambda i: (i, 0)),
                  row64,
                  pl.BlockSpec((1, 64), lambda i: (0, 0))],
        out_specs=row64,
        out_shape=jax.ShapeDtypeStruct((N, 64), jnp.float32),
    )(agg2, ca, cb, r2, b2.reshape(1, 64))


def kernel(video, audio, title, edge_index, user, Wv, Wa, Wt, b_f, id_emb,
           W1l, W1r, b1, W2l, W2r, b2):
    src = edge_index[0]
    dst = edge_index[1]
    zpad = jnp.zeros((EP - E,), jnp.int32)
    srcpad = jnp.concatenate([src, zpad])
    srcp8 = ((srcpad * 8)[None, :]
             + jnp.arange(8, dtype=jnp.int32)[:, None]).reshape(
                 8 * EROWS, IDXW)
    srcp4 = ((srcpad * 4)[None, :]
             + jnp.arange(4, dtype=jnp.int32)[:, None]).reshape(
                 4 * EROWS, IDXW)
    dstp = jnp.concatenate(
        [dst, jnp.full((EP - E,), N, jnp.int32)]).reshape(EROWS, IDXW)

    md = _tc_fuse(video, audio, title, id_emb, Wv, Wa, Wt, b_f)
    x = jnp.concatenate([user, md], axis=0)

    cnt2 = _sc_count(dstp)
    ca = cnt2[:N, None]
    cb = cnt2[NCNT:NCNT + N, None]
    agg = _sc_agg(8, x.reshape(N * 8, W), srcp8, dstp)

    h2, r2 = _tc_layer1(agg, x, ca, cb, W1l, W1r, b1, W2l, W2r)

    agg2 = _sc_agg(4, h2.reshape(N * 4, W), srcp4, dstp)

    return _tc_final(agg2, ca, cb, r2, b2)
